# baseline (speedup 1.0000x reference)
import functools

import jax
import jax.numpy as jnp
from jax import lax
from jax.experimental import pallas as pl
from jax.experimental.pallas import tpu as pltpu
from jax.experimental.pallas import tpu_sc as plsc


_NB = 4
_NW = 32
_SEQ_PAD = 56
_LANE = 128


@functools.partial(jax.jit, static_argnums=(2,))
def _sc_gather(token_ids, weight_pad, d):
    bsz, seq = token_ids.shape
    b_per_w = bsz // _NW
    n_rounds = b_per_w // _NB
    half = n_rounds // 2

    mesh = plsc.VectorSubcoreMesh(core_axis_name="c", subcore_axis_name="s")

    @functools.partial(
        pl.kernel,
        out_type=jax.ShapeDtypeStruct((bsz * _SEQ_PAD, _LANE), jnp.float32),
        mesh=mesh,
        scratch_types=[
            pltpu.VMEM((b_per_w, seq), jnp.int32),
            pltpu.VMEM((_NB, seq, _LANE), jnp.float32),
            pltpu.VMEM((_NB, seq, _LANE), jnp.float32),
            pltpu.VMEM((_NB, seq, d), jnp.float32),
            pltpu.VMEM((_NB, seq, d), jnp.float32),
            pltpu.SemaphoreType.DMA,
            pltpu.SemaphoreType.DMA,
            pltpu.SemaphoreType.DMA,
            pltpu.SemaphoreType.DMA,
        ],
        compiler_params=pltpu.CompilerParams(use_tc_tiling_on_sc=False),
    )
    def k(weight_hbm, idx_hbm, out_hbm, idx_v, buf0, buf1, cb0, cb1,
          gs0, gs1, os0, os1):
        num_cores = mesh.num_cores
        wid = lax.axis_index("s") * num_cores + lax.axis_index("c")
        bat0 = wid * b_per_w
        pltpu.sync_copy(idx_hbm.at[pl.ds(bat0, b_per_w)], idx_v)

        def fire(g, buf, sem):
            for j in range(_NB):
                pltpu.async_copy(
                    weight_hbm.at[idx_v.at[g * _NB + j]],
                    buf.at[j],
                    sem,
                )

        def drain_gather(buf, sem):
            for j in range(_NB):
                pltpu.make_async_copy(
                    weight_hbm.at[pl.ds(0, seq)], buf.at[j], sem).wait()

        def compact(buf, cb):
            @pl.loop(0, seq)
            def _p(p):
                for j in range(_NB):
                    for c in range(0, d, 16):
                        cb[j, p, pl.ds(c, 16)] = buf[j, p, pl.ds(c, 16)]

        def out_copies(g, cb, sem):
            for j in range(_NB):
                row0 = (bat0 + g * _NB + j) * _SEQ_PAD
                pltpu.async_copy(
                    cb.at[j],
                    out_hbm.at[pl.ds(row0, seq), pl.ds(0, d)],
                    sem,
                )

        def wait_out(cb, sem):
            for j in range(_NB):
                pltpu.make_async_copy(
                    cb.at[j],
                    out_hbm.at[pl.ds(0, seq), pl.ds(0, d)],
                    sem).wait()

        fire(0, buf0, gs0)

        @pl.loop(0, half)
        def _h(h):
            a = 2 * h
            b = a + 1

            fire(b, buf1, gs1)

            @pl.when(h > 0)
            def _():
                wait_out(cb0, os0)

            drain_gather(buf0, gs0)
            compact(buf0, cb0)

            @pl.when(h + 1 < half)
            def _():
                fire(a + 2, buf0, gs0)

            out_copies(a, cb0, os0)

            @pl.when(h > 0)
            def _():
                wait_out(cb1, os1)

            drain_gather(buf1, gs1)
            compact(buf1, cb1)
            out_copies(b, cb1, os1)

        wait_out(cb0, os0)
        wait_out(cb1, os1)

    return k(weight_pad, token_ids)


def kernel(token_ids, weight):
    bsz, seq = token_ids.shape
    v, d = weight.shape
    idx = token_ids.astype(jnp.int32)
    assert bsz % (_NW * _NB * 2) == 0 and seq <= _SEQ_PAD and d <= _LANE
    assert d % 16 == 0
    wpad = jnp.pad(weight, ((0, 0), (0, _LANE - d)))
    out = _sc_gather(idx, wpad, d)
    return out.reshape(bsz, _SEQ_PAD, _LANE)[:, :seq, :d]


# speedup vs baseline: 1.1820x; 1.1820x over previous
import functools

import jax
import jax.numpy as jnp
from jax import lax
from jax.experimental import pallas as pl
from jax.experimental.pallas import tpu as pltpu
from jax.experimental.pallas import tpu_sc as plsc


_NB = 16
_NW = 32
_SEQ_PAD = 56


@jax.jit
def _sc_gather(token_ids, weight):
    bsz, seq = token_ids.shape
    d = weight.shape[1]
    b_per_w = bsz // _NW
    n_rounds = b_per_w // _NB
    half = n_rounds // 2

    mesh = plsc.VectorSubcoreMesh(core_axis_name="c", subcore_axis_name="s")

    @functools.partial(
        pl.kernel,
        out_type=jax.ShapeDtypeStruct((bsz * _SEQ_PAD, 128), jnp.float32),
        mesh=mesh,
        scratch_types=[
            pltpu.VMEM((b_per_w, seq), jnp.int32),
            pltpu.VMEM((_NB, seq, d), jnp.float32),
            pltpu.VMEM((_NB, seq, d), jnp.float32),
            pltpu.SemaphoreType.DMA,
            pltpu.SemaphoreType.DMA,
            pltpu.SemaphoreType.DMA,
            pltpu.SemaphoreType.DMA,
        ],
        compiler_params=pltpu.CompilerParams(use_tc_tiling_on_sc=False),
    )
    def k(weight_hbm, idx_hbm, out_hbm, idx_v, buf0, buf1, gs0, gs1, os0, os1):
        num_cores = mesh.num_cores
        wid = lax.axis_index("s") * num_cores + lax.axis_index("c")
        bat0 = wid * b_per_w
        pltpu.sync_copy(idx_hbm.at[pl.ds(bat0, b_per_w)], idx_v)

        def fire(g, buf, sem):
            for j in range(_NB):
                pltpu.async_copy(
                    weight_hbm.at[idx_v.at[g * _NB + j]],
                    buf.at[j],
                    sem,
                )

        def drain_gather(buf, sem):
            for j in range(_NB):
                pltpu.make_async_copy(
                    weight_hbm.at[pl.ds(0, seq)], buf.at[j], sem).wait()

        def out_copies(g, buf, sem):
            for j in range(_NB):
                row0 = (bat0 + g * _NB + j) * _SEQ_PAD
                pltpu.async_copy(
                    buf.at[j],
                    out_hbm.at[pl.ds(row0, seq), pl.ds(0, d)],
                    sem,
                )

        def wait_out(buf, sem):
            for j in range(_NB):
                pltpu.make_async_copy(
                    buf.at[j],
                    out_hbm.at[pl.ds(0, seq), pl.ds(0, d)],
                    sem).wait()

        fire(0, buf0, gs0)

        @pl.loop(0, half)
        def _h(h):
            a = 2 * h
            b = a + 1

            @pl.when(h > 0)
            def _():
                wait_out(buf1, os1)

            fire(b, buf1, gs1)
            drain_gather(buf0, gs0)
            out_copies(a, buf0, os0)
            drain_gather(buf1, gs1)
            wait_out(buf0, os0)

            @pl.when(h + 1 < half)
            def _():
                fire(a + 2, buf0, gs0)

            out_copies(b, buf1, os1)

        wait_out(buf1, os1)

    return k(weight, token_ids)


def kernel(token_ids, weight):
    bsz, seq = token_ids.shape
    d = weight.shape[1]
    idx = token_ids.astype(jnp.int32)
    assert bsz % (_NW * _NB * 2) == 0 and seq <= _SEQ_PAD
    out = _sc_gather(idx, weight)
    return out.reshape(bsz, _SEQ_PAD, 128)[:, :seq, :d]
